# Initial kernel scaffold; baseline (speedup 1.0000x reference)
import jax
import jax.numpy as jnp
from jax.experimental import pallas as pl


def kernel(token_ids, weight):
    raise NotImplementedError("write your pallas kernel here")


# speedup vs baseline: 6.3206x; 6.3206x over previous
import functools

import jax
import jax.numpy as jnp
from jax import lax
from jax.experimental import pallas as pl
from jax.experimental.pallas import tpu as pltpu
from jax.experimental.pallas import tpu_sc as plsc

_NC = 2
_NS = 16
_NW = _NC * _NS
_C = 128


@functools.lru_cache(maxsize=None)
def _make_embed(B: int, D: int):
    bpw = B // _NW
    nchunks = bpw // _C
    mesh = plsc.VectorSubcoreMesh(core_axis_name="c", subcore_axis_name="s")

    @functools.partial(
        pl.kernel,
        mesh=mesh,
        out_type=jax.ShapeDtypeStruct((B, D), jnp.float32),
        scratch_types=[
            pltpu.VMEM((nchunks, _C), jnp.int32),
            pltpu.VMEM((_C, D), jnp.float32),
            pltpu.SemaphoreType.DMA,
        ],
    )
    def embed(idx_hbm, table_hbm, out_hbm, idx_v, rows_v, sem):
        wid = lax.axis_index("s") * _NC + lax.axis_index("c")
        base = wid * bpw
        pltpu.sync_copy(idx_hbm.at[pl.ds(wid * nchunks, nchunks)], idx_v)

        def body(j, carry):
            pltpu.async_copy(table_hbm.at[idx_v.at[j]], rows_v, sem).wait()
            pltpu.sync_copy(rows_v, out_hbm.at[pl.ds(base + j * _C, _C)])
            return carry

        lax.fori_loop(0, nchunks, body, 0)

    return embed


def kernel(token_ids, weight):
    S, T = token_ids.shape
    D = weight.shape[1]
    B = S * T
    idx = token_ids.reshape(B // _C, _C).astype(jnp.int32)
    out = _make_embed(B, D)(idx, weight)
    return out.reshape(S, T, D)


# speedup vs baseline: 9.0932x; 1.4387x over previous
import functools

import jax
import jax.numpy as jnp
from jax import lax
from jax.experimental import pallas as pl
from jax.experimental.pallas import tpu as pltpu
from jax.experimental.pallas import tpu_sc as plsc

_NC = 2
_NS = 16
_NW = _NC * _NS
_C = 128
_NBUF = 4


@functools.lru_cache(maxsize=None)
def _make_embed(B: int, D: int):
    bpw = B // _NW
    nchunks = bpw // _C
    mesh = plsc.VectorSubcoreMesh(core_axis_name="c", subcore_axis_name="s")

    @functools.partial(
        pl.kernel,
        mesh=mesh,
        out_type=jax.ShapeDtypeStruct((B, D), jnp.float32),
        scratch_types=(
            [pltpu.VMEM((nchunks, _C), jnp.int32)]
            + [pltpu.VMEM((_C, D), jnp.float32) for _ in range(_NBUF)]
            + [pltpu.SemaphoreType.DMA for _ in range(2 * _NBUF)]
        ),
    )
    def embed(idx_hbm, table_hbm, out_hbm, idx_v, *bufs_and_sems):
        bufs = bufs_and_sems[:_NBUF]
        gsem = bufs_and_sems[_NBUF:2 * _NBUF]
        ssem = bufs_and_sems[2 * _NBUF:]
        wid = lax.axis_index("s") * _NC + lax.axis_index("c")
        base = wid * bpw
        pltpu.sync_copy(idx_hbm.at[pl.ds(wid * nchunks, nchunks)], idx_v)

        def gather(j, b):
            return pltpu.make_async_copy(
                table_hbm.at[idx_v.at[j]], bufs[b], gsem[b])

        def store(j, b):
            return pltpu.make_async_copy(
                bufs[b], out_hbm.at[pl.ds(base + j * _C, _C)], ssem[b])

        for b in range(_NBUF):
            gather(b, b).start()

        def body(r, carry):
            k = r * _NBUF
            for b in range(_NBUF):
                j = k + b
                gather(j, b).wait()
                store(j, b).start()
            for b in range(_NBUF):
                j2 = k + _NBUF + b

                @pl.when(j2 < nchunks)
                def _():
                    store(k + b, b).wait()
                    gather(j2, b).start()
            return carry

        lax.fori_loop(0, nchunks // _NBUF, body, 0)

        for b in range(_NBUF):
            store(nchunks - _NBUF + b, b).wait()

    return embed


def kernel(token_ids, weight):
    S, T = token_ids.shape
    D = weight.shape[1]
    B = S * T
    idx = token_ids.reshape(B // _C, _C).astype(jnp.int32)
    out = _make_embed(B, D)(idx, weight)
    return out.reshape(S, T, D)
